# baseline (speedup 1.0000x reference)
import functools

import jax
import jax.numpy as jnp
from jax import lax
from jax.experimental import pallas as pl
from jax.experimental.pallas import tpu as pltpu
from jax.experimental.pallas import tpu_sc as plsc

_LANES = 16
_NUM_CORES = 2
_NUM_SUBCORES = 16
_NW = _NUM_CORES * _NUM_SUBCORES


def _build_sc_call(B, C, NF, L):
    L2 = 2 * L
    bc = B * C
    assert bc % _NW == 0
    bc_per_w = bc // _NW
    CHUNK = 4096
    assert L2 % CHUNK == 0
    zslot = NF

    mesh = plsc.VectorSubcoreMesh(core_axis_name="c", subcore_axis_name="s")

    @functools.partial(
        pl.kernel,
        out_type=jax.ShapeDtypeStruct((_NW * _LANES,), jnp.float32),
        mesh=mesh,
        compiler_params=pltpu.CompilerParams(needs_layout_passes=False),
        scratch_types=[
            pltpu.VMEM((L2,), jnp.int32),
            pltpu.VMEM((NF + _LANES,), jnp.float32),
            pltpu.VMEM((NF + _LANES,), jnp.float32),
            pltpu.VMEM((NF + _LANES,), jnp.float32),
            pltpu.VMEM((NF + _LANES,), jnp.float32),
            pltpu.VMEM((2 * NF,), jnp.float32),
            pltpu.VMEM((2 * NF,), jnp.float32),
            pltpu.VMEM((2 * L,), jnp.float32),
            pltpu.VMEM((2 * L,), jnp.float32),
            pltpu.VMEM((2 * CHUNK,), jnp.float32),
            pltpu.VMEM((2 * CHUNK,), jnp.float32),
            pltpu.VMEM((_LANES,), jnp.float32),
        ],
    )
    def sc_call(i_f, i_s, t_f, t_s, xi0, xi1, ks0, ks1, out,
                idxp, fre, fim, tre, tim, sta, stb, xis, kss, sci, sct, accv):
        cid = lax.axis_index("c")
        sid = lax.axis_index("s")
        wid = sid * _NUM_CORES + cid

        iota2 = lax.iota(jnp.int32, _LANES) * 2
        z16f = jnp.zeros((_LANES,), jnp.float32)

        fre[pl.ds(NF, _LANES)] = z16f
        fim[pl.ds(NF, _LANES)] = z16f
        tre[pl.ds(NF, _LANES)] = z16f
        tim[pl.ds(NF, _LANES)] = z16f

        for half, (xi, ks) in enumerate(((xi0, ks0), (xi1, ks1))):
            pltpu.sync_copy(xi, xis)
            pltpu.sync_copy(ks, kss)

            def pre_body(j, carry, half=half):
                for u in range(2):
                    lj = j * (2 * _LANES) + u * _LANES
                    le = 2 * lj + iota2
                    x0 = plsc.bitcast(plsc.load_gather(xis, [le]), jnp.int32)
                    x1 = plsc.bitcast(plsc.load_gather(xis, [le + 1]),
                                      jnp.int32)
                    k0 = plsc.load_gather(kss, [le])
                    k1 = plsc.load_gather(kss, [le + 1])
                    keep = jnp.logical_and(k0 <= 0.0, k1 <= 0.0)
                    i0m = jnp.where(keep, x0, zslot)
                    packed = jnp.bitwise_or(i0m, jnp.left_shift(x1, 16))
                    idxp[pl.ds(half * L + lj, _LANES)] = packed
                return carry

            lax.fori_loop(0, L // (2 * _LANES), pre_body, 0)

        def per_pair(j, acc):
            p = wid * bc_per_w + j

            pltpu.sync_copy(i_f.at[pl.ds(p * (2 * NF), 2 * NF)], sta)
            pltpu.sync_copy(t_f.at[pl.ds(p * (2 * NF), 2 * NF)], stb)

            def tbl_body(n, carry):
                for u in range(2):
                    nj = n * (2 * _LANES) + u * _LANES
                    ne = 2 * nj + iota2
                    ife = plsc.load_gather(sta, [ne])
                    ifo = plsc.load_gather(sta, [ne + 1])
                    tfe = plsc.load_gather(stb, [ne])
                    tfo = plsc.load_gather(stb, [ne + 1])
                    sl = pl.ds(nj, _LANES)
                    fre[sl] = ife - tfe
                    fim[sl] = ifo - tfo
                    tre[sl] = tfe
                    tim[sl] = tfo
                return carry

            lax.fori_loop(0, NF // (2 * _LANES), tbl_body, 0)

            def per_chunk(ch, acc):
                s_off = ch * CHUNK
                src = p * (2 * L2) + 2 * s_off
                pltpu.sync_copy(i_s.at[pl.ds(src, 2 * CHUNK)], sci)
                pltpu.sync_copy(t_s.at[pl.ds(src, 2 * CHUNK)], sct)

                def inner(i, acc):
                    parts = []
                    for u in range(4):
                        li = i * (4 * _LANES) + u * _LANES
                        le = 2 * li + iota2
                        pk = idxp[pl.ds(s_off + li, _LANES)]
                        i0 = jnp.bitwise_and(pk, 0xFFFF)
                        i1 = lax.shift_right_logical(pk, 16)
                        fr0 = plsc.load_gather(fre, [i0])
                        fi0 = plsc.load_gather(fim, [i0])
                        tr0 = plsc.load_gather(tre, [i0])
                        ti0 = plsc.load_gather(tim, [i0])
                        fr1 = plsc.load_gather(fre, [i1])
                        fi1 = plsc.load_gather(fim, [i1])
                        tr1 = plsc.load_gather(tre, [i1])
                        ti1 = plsc.load_gather(tim, [i1])
                        sre = (plsc.load_gather(sci, [le])
                               - plsc.load_gather(sct, [le]))
                        sim = (plsc.load_gather(sci, [le + 1])
                               - plsc.load_gather(sct, [le + 1]))
                        ere = fr0 * tr1 + fi0 * ti1 + tr0 * fr1 + ti0 * fi1
                        eim = fi0 * tr1 - fr0 * ti1 + ti0 * fr1 - tr0 * fi1
                        gre = sre - ere
                        gim = sim - eim
                        parts.append(gre * gre + gim * gim)
                    return acc + ((parts[0] + parts[1])
                                  + (parts[2] + parts[3]))

                return lax.fori_loop(0, CHUNK // (4 * _LANES), inner, acc)

            return lax.fori_loop(0, L2 // CHUNK, per_chunk, acc)

        acc = lax.fori_loop(0, bc_per_w, per_pair, z16f)
        accv[...] = acc
        pltpu.sync_copy(accv, out.at[pl.ds(wid * _LANES, _LANES)])

    return sc_call


@jax.jit
def kernel(i_f, i_s, t_f, t_s, xi_idx0, xi_idx1, ks0, ks1):
    B, C, NF, _ = i_f.shape
    L = xi_idx0.shape[0]
    sc_call = _build_sc_call(B, C, NF, L)
    partials = sc_call(
        i_f.reshape(-1),
        i_s.reshape(-1),
        t_f.reshape(-1),
        t_s.reshape(-1),
        lax.bitcast_convert_type(xi_idx0, jnp.float32).reshape(-1),
        lax.bitcast_convert_type(xi_idx1, jnp.float32).reshape(-1),
        ks0.reshape(-1),
        ks1.reshape(-1),
    )
    n = B * C * 2 * L
    return jnp.sum(partials) * jnp.float32(1.0 / n)


# speedup vs baseline: 27.0892x; 27.0892x over previous
import functools

import jax
import jax.numpy as jnp
from jax import lax
from jax.experimental import pallas as pl
from jax.experimental.pallas import tpu as pltpu
from jax.experimental.pallas import tpu_sc as plsc

_LANES = 16
_NUM_CORES = 2
_NUM_SUBCORES = 16
_NW = _NUM_CORES * _NUM_SUBCORES


def _build_sc_call(B, C, NF, L):
    L2 = 2 * L
    bc = B * C
    assert bc % _NW == 0
    bc_per_w = bc // _NW
    CHUNK = 4096
    assert L2 % CHUNK == 0
    nch = L2 // CHUNK
    zslot = NF

    mesh = plsc.VectorSubcoreMesh(core_axis_name="c", subcore_axis_name="s")

    @functools.partial(
        pl.kernel,
        out_type=jax.ShapeDtypeStruct((_NW * _LANES,), jnp.float32),
        mesh=mesh,
        compiler_params=pltpu.CompilerParams(needs_layout_passes=False),
        scratch_types=[
            pltpu.VMEM((L2,), jnp.int32),
            pltpu.VMEM((NF + _LANES,), jnp.float32),
            pltpu.VMEM((NF + _LANES,), jnp.float32),
            pltpu.VMEM((NF + _LANES,), jnp.float32),
            pltpu.VMEM((NF + _LANES,), jnp.float32),
            pltpu.VMEM((2 * NF,), jnp.float32),
            pltpu.VMEM((2 * NF,), jnp.float32),
            pltpu.VMEM((2 * L,), jnp.float32),
            pltpu.VMEM((2 * L,), jnp.float32),
            pltpu.VMEM((2 * CHUNK,), jnp.float32),
            pltpu.VMEM((2 * CHUNK,), jnp.float32),
            pltpu.VMEM((2 * CHUNK,), jnp.float32),
            pltpu.VMEM((2 * CHUNK,), jnp.float32),
            pltpu.VMEM((_LANES,), jnp.float32),
            pltpu.SemaphoreType.DMA,
            pltpu.SemaphoreType.DMA,
            pltpu.SemaphoreType.DMA,
        ],
    )
    def sc_call(i_f, i_s, t_f, t_s, xi0, xi1, ks0, ks1, out,
                idxp, fre, fim, tre, tim, sta, stb, xis, kss,
                sci0, sci1, sct0, sct1, accv, sem_f, sem0, sem1):
        cid = lax.axis_index("c")
        sid = lax.axis_index("s")
        wid = sid * _NUM_CORES + cid

        scis = (sci0, sci1)
        scts = (sct0, sct1)
        sems = (sem0, sem1)

        iota2 = lax.iota(jnp.int32, _LANES) * 2
        z16f = jnp.zeros((_LANES,), jnp.float32)

        fre[pl.ds(NF, _LANES)] = z16f
        fim[pl.ds(NF, _LANES)] = z16f
        tre[pl.ds(NF, _LANES)] = z16f
        tim[pl.ds(NF, _LANES)] = z16f

        for half, (xi, ks) in enumerate(((xi0, ks0), (xi1, ks1))):
            pltpu.sync_copy(xi, xis)
            pltpu.sync_copy(ks, kss)

            def pre_body(j, carry, half=half):
                for u in range(2):
                    lj = j * (2 * _LANES) + u * _LANES
                    le = 2 * lj + iota2
                    x0 = plsc.bitcast(plsc.load_gather(xis, [le]), jnp.int32)
                    x1 = plsc.bitcast(plsc.load_gather(xis, [le + 1]),
                                      jnp.int32)
                    k0 = plsc.load_gather(kss, [le])
                    k1 = plsc.load_gather(kss, [le + 1])
                    keep = jnp.logical_and(k0 <= 0.0, k1 <= 0.0)
                    i0m = jnp.where(keep, x0, zslot)
                    packed = jnp.bitwise_or(i0m, jnp.left_shift(x1, 16))
                    idxp[pl.ds(half * L + lj, _LANES)] = packed
                return carry

            lax.fori_loop(0, L // (2 * _LANES), pre_body, 0)

        def per_pair(j, acc):
            p = wid * bc_per_w + j
            b = p // C
            c = p % C

            d_f0 = pltpu.async_copy(i_f.at[b, c], sta, sem_f)
            d_f1 = pltpu.async_copy(t_f.at[b, c], stb, sem_f)

            descs = {}

            def start_chunk(ch):
                par = ch % 2
                sl = pl.ds(2 * ch * CHUNK, 2 * CHUNK)
                descs[ch] = (
                    pltpu.async_copy(i_s.at[b, c, sl], scis[par], sems[par]),
                    pltpu.async_copy(t_s.at[b, c, sl], scts[par], sems[par]),
                )

            start_chunk(0)
            start_chunk(1)
            d_f0.wait()
            d_f1.wait()

            def tbl_body(n, carry):
                for u in range(2):
                    nj = n * (2 * _LANES) + u * _LANES
                    ne = 2 * nj + iota2
                    ife = plsc.load_gather(sta, [ne])
                    ifo = plsc.load_gather(sta, [ne + 1])
                    tfe = plsc.load_gather(stb, [ne])
                    tfo = plsc.load_gather(stb, [ne + 1])
                    sl = pl.ds(nj, _LANES)
                    fre[sl] = ife - tfe
                    fim[sl] = ifo - tfo
                    tre[sl] = tfe
                    tim[sl] = tfo
                return carry

            lax.fori_loop(0, NF // (2 * _LANES), tbl_body, 0)

            for ch in range(nch):
                par = ch % 2
                sci, sct = scis[par], scts[par]
                s_off = ch * CHUNK
                descs[ch][0].wait()
                descs[ch][1].wait()

                def inner(i, acc, sci=sci, sct=sct, s_off=s_off):
                    parts = []
                    for u in range(4):
                        li = i * (4 * _LANES) + u * _LANES
                        le = 2 * li + iota2
                        pk = idxp[pl.ds(s_off + li, _LANES)]
                        i0 = jnp.bitwise_and(pk, 0xFFFF)
                        i1 = lax.shift_right_logical(pk, 16)
                        fr0 = plsc.load_gather(fre, [i0])
                        fi0 = plsc.load_gather(fim, [i0])
                        tr0 = plsc.load_gather(tre, [i0])
                        ti0 = plsc.load_gather(tim, [i0])
                        fr1 = plsc.load_gather(fre, [i1])
                        fi1 = plsc.load_gather(fim, [i1])
                        tr1 = plsc.load_gather(tre, [i1])
                        ti1 = plsc.load_gather(tim, [i1])
                        sre = (plsc.load_gather(sci, [le])
                               - plsc.load_gather(sct, [le]))
                        sim = (plsc.load_gather(sci, [le + 1])
                               - plsc.load_gather(sct, [le + 1]))
                        ere = fr0 * tr1 + fi0 * ti1 + tr0 * fr1 + ti0 * fi1
                        eim = fi0 * tr1 - fr0 * ti1 + ti0 * fr1 - tr0 * fi1
                        gre = sre - ere
                        gim = sim - eim
                        parts.append(gre * gre + gim * gim)
                    return acc + ((parts[0] + parts[1])
                                  + (parts[2] + parts[3]))

                acc = lax.fori_loop(0, CHUNK // (4 * _LANES), inner, acc)
                if ch + 2 < nch:
                    start_chunk(ch + 2)

            return acc

        acc = lax.fori_loop(0, bc_per_w, per_pair, z16f)
        accv[...] = acc
        pltpu.sync_copy(accv, out.at[pl.ds(wid * _LANES, _LANES)])

    return sc_call


@jax.jit
def kernel(i_f, i_s, t_f, t_s, xi_idx0, xi_idx1, ks0, ks1):
    B, C, NF, _ = i_f.shape
    L = xi_idx0.shape[0]
    sc_call = _build_sc_call(B, C, NF, L)
    partials = sc_call(
        i_f.reshape(B, C, 2 * NF),
        i_s.reshape(B, C, 4 * L),
        t_f.reshape(B, C, 2 * NF),
        t_s.reshape(B, C, 4 * L),
        lax.bitcast_convert_type(xi_idx0, jnp.float32).reshape(2 * L),
        lax.bitcast_convert_type(xi_idx1, jnp.float32).reshape(2 * L),
        ks0.reshape(2 * L),
        ks1.reshape(2 * L),
    )
    n = B * C * 2 * L
    return jnp.sum(partials) * jnp.float32(1.0 / n)


# speedup vs baseline: 29.2254x; 1.0789x over previous
import functools

import jax
import jax.numpy as jnp
from jax import lax
from jax.experimental import pallas as pl
from jax.experimental.pallas import tpu as pltpu
from jax.experimental.pallas import tpu_sc as plsc

_LANES = 16
_NUM_CORES = 2
_NUM_SUBCORES = 16
_NW = _NUM_CORES * _NUM_SUBCORES


def _build_sc_call(B, C, NF, L):
    L2 = 2 * L
    bc = B * C
    assert bc % _NW == 0
    bc_per_w = bc // _NW
    CHUNK = 4096
    assert L2 % CHUNK == 0
    nch = L2 // CHUNK
    zslot = NF

    mesh = plsc.VectorSubcoreMesh(core_axis_name="c", subcore_axis_name="s")

    @functools.partial(
        pl.kernel,
        out_type=jax.ShapeDtypeStruct((_NW * _LANES,), jnp.float32),
        mesh=mesh,
        compiler_params=pltpu.CompilerParams(needs_layout_passes=False),
        scratch_types=[
            pltpu.VMEM((L2,), jnp.int32),
            pltpu.VMEM((NF + _LANES,), jnp.float32),
            pltpu.VMEM((NF + _LANES,), jnp.float32),
            pltpu.VMEM((NF + _LANES,), jnp.float32),
            pltpu.VMEM((NF + _LANES,), jnp.float32),
            pltpu.VMEM((2 * NF,), jnp.float32),
            pltpu.VMEM((2 * NF,), jnp.float32),
            pltpu.VMEM((2 * L,), jnp.float32),
            pltpu.VMEM((2 * L,), jnp.float32),
            pltpu.VMEM((2 * CHUNK,), jnp.float32),
            pltpu.VMEM((2 * CHUNK,), jnp.float32),
            pltpu.VMEM((2 * CHUNK,), jnp.float32),
            pltpu.VMEM((2 * CHUNK,), jnp.float32),
            pltpu.VMEM((_LANES,), jnp.float32),
            pltpu.SemaphoreType.DMA,
            pltpu.SemaphoreType.DMA,
            pltpu.SemaphoreType.DMA,
        ],
    )
    def sc_call(cat, xi0, xi1, ks0, ks1, out,
                idxp, fre, fim, tre, tim, sta, stb, xis, kss,
                sci0, sci1, sct0, sct1, accv, sem_f, sem0, sem1):
        off_tf = 2 * NF
        off_is = 4 * NF
        off_ts = 4 * NF + 2 * L2
        cid = lax.axis_index("c")
        sid = lax.axis_index("s")
        wid = sid * _NUM_CORES + cid

        scis = (sci0, sci1)
        scts = (sct0, sct1)
        sems = (sem0, sem1)

        iota2 = lax.iota(jnp.int32, _LANES) * 2
        z16f = jnp.zeros((_LANES,), jnp.float32)

        fre[pl.ds(NF, _LANES)] = z16f
        fim[pl.ds(NF, _LANES)] = z16f
        tre[pl.ds(NF, _LANES)] = z16f
        tim[pl.ds(NF, _LANES)] = z16f

        for half, (xi, ks) in enumerate(((xi0, ks0), (xi1, ks1))):
            pltpu.sync_copy(xi, xis)
            pltpu.sync_copy(ks, kss)

            def pre_body(j, carry, half=half):
                for u in range(2):
                    lj = j * (2 * _LANES) + u * _LANES
                    le = 2 * lj + iota2
                    x0 = plsc.bitcast(plsc.load_gather(xis, [le]), jnp.int32)
                    x1 = plsc.bitcast(plsc.load_gather(xis, [le + 1]),
                                      jnp.int32)
                    k0 = plsc.load_gather(kss, [le])
                    k1 = plsc.load_gather(kss, [le + 1])
                    keep = jnp.logical_and(k0 <= 0.0, k1 <= 0.0)
                    i0m = jnp.where(keep, x0, zslot)
                    packed = jnp.bitwise_or(i0m, jnp.left_shift(x1, 16))
                    idxp[pl.ds(half * L + lj, _LANES)] = packed
                return carry

            lax.fori_loop(0, L // (2 * _LANES), pre_body, 0)

        def per_pair(j, acc):
            p = wid * bc_per_w + j
            b = p // C
            c = p % C

            d_f0 = pltpu.async_copy(
                cat.at[b, c, pl.ds(0, 2 * NF)], sta, sem_f)
            d_f1 = pltpu.async_copy(
                cat.at[b, c, pl.ds(off_tf, 2 * NF)], stb, sem_f)

            descs = {}

            def start_chunk(ch):
                par = ch % 2
                descs[ch] = (
                    pltpu.async_copy(
                        cat.at[b, c, pl.ds(off_is + 2 * ch * CHUNK,
                                           2 * CHUNK)],
                        scis[par], sems[par]),
                    pltpu.async_copy(
                        cat.at[b, c, pl.ds(off_ts + 2 * ch * CHUNK,
                                           2 * CHUNK)],
                        scts[par], sems[par]),
                )

            start_chunk(0)
            start_chunk(1)
            d_f0.wait()
            d_f1.wait()

            def tbl_body(n, carry):
                for u in range(2):
                    nj = n * (2 * _LANES) + u * _LANES
                    ne = 2 * nj + iota2
                    ife = plsc.load_gather(sta, [ne])
                    ifo = plsc.load_gather(sta, [ne + 1])
                    tfe = plsc.load_gather(stb, [ne])
                    tfo = plsc.load_gather(stb, [ne + 1])
                    sl = pl.ds(nj, _LANES)
                    fre[sl] = ife - tfe
                    fim[sl] = ifo - tfo
                    tre[sl] = tfe
                    tim[sl] = tfo
                return carry

            lax.fori_loop(0, NF // (2 * _LANES), tbl_body, 0)

            for ch in range(nch):
                par = ch % 2
                sci, sct = scis[par], scts[par]
                s_off = ch * CHUNK
                descs[ch][0].wait()
                descs[ch][1].wait()

                def inner(i, acc, sci=sci, sct=sct, s_off=s_off):
                    parts = []
                    for u in range(4):
                        li = i * (4 * _LANES) + u * _LANES
                        le = 2 * li + iota2
                        pk = idxp[pl.ds(s_off + li, _LANES)]
                        i0 = jnp.bitwise_and(pk, 0xFFFF)
                        i1 = lax.shift_right_logical(pk, 16)
                        fr0 = plsc.load_gather(fre, [i0])
                        fi0 = plsc.load_gather(fim, [i0])
                        tr0 = plsc.load_gather(tre, [i0])
                        ti0 = plsc.load_gather(tim, [i0])
                        fr1 = plsc.load_gather(fre, [i1])
                        fi1 = plsc.load_gather(fim, [i1])
                        tr1 = plsc.load_gather(tre, [i1])
                        ti1 = plsc.load_gather(tim, [i1])
                        sre = (plsc.load_gather(sci, [le])
                               - plsc.load_gather(sct, [le]))
                        sim = (plsc.load_gather(sci, [le + 1])
                               - plsc.load_gather(sct, [le + 1]))
                        ere = fr0 * tr1 + fi0 * ti1 + tr0 * fr1 + ti0 * fi1
                        eim = fi0 * tr1 - fr0 * ti1 + ti0 * fr1 - tr0 * fi1
                        gre = sre - ere
                        gim = sim - eim
                        parts.append(gre * gre + gim * gim)
                    return acc + ((parts[0] + parts[1])
                                  + (parts[2] + parts[3]))

                acc = lax.fori_loop(0, CHUNK // (4 * _LANES), inner, acc)
                if ch + 2 < nch:
                    start_chunk(ch + 2)

            return acc

        acc = lax.fori_loop(0, bc_per_w, per_pair, z16f)
        accv[...] = acc
        pltpu.sync_copy(accv, out.at[pl.ds(wid * _LANES, _LANES)])

    return sc_call


@jax.jit
def kernel(i_f, i_s, t_f, t_s, xi_idx0, xi_idx1, ks0, ks1):
    B, C, NF, _ = i_f.shape
    L = xi_idx0.shape[0]
    sc_call = _build_sc_call(B, C, NF, L)
    cat = jnp.concatenate(
        [i_f.reshape(B, C, 2 * NF), t_f.reshape(B, C, 2 * NF),
         i_s.reshape(B, C, 4 * L), t_s.reshape(B, C, 4 * L)], axis=-1)
    partials = sc_call(
        cat,
        lax.bitcast_convert_type(xi_idx0, jnp.float32).reshape(2 * L),
        lax.bitcast_convert_type(xi_idx1, jnp.float32).reshape(2 * L),
        ks0.reshape(2 * L),
        ks1.reshape(2 * L),
    )
    n = B * C * 2 * L
    return jnp.sum(partials) * jnp.float32(1.0 / n)


# speedup vs baseline: 31.3947x; 1.0742x over previous
import functools

import jax
import jax.numpy as jnp
from jax import lax
from jax.experimental import pallas as pl
from jax.experimental.pallas import tpu as pltpu
from jax.experimental.pallas import tpu_sc as plsc

_LANES = 16
_NUM_CORES = 2
_NUM_SUBCORES = 16
_NW = _NUM_CORES * _NUM_SUBCORES


def _build_sc_call(B, C, NF, L):
    L2 = 2 * L
    bc = B * C
    assert bc % _NW == 0
    bc_per_w = bc // _NW
    CHUNK = 4096
    assert L2 % CHUNK == 0
    nch = L2 // CHUNK
    zslot = NF

    mesh = plsc.VectorSubcoreMesh(core_axis_name="c", subcore_axis_name="s")

    @functools.partial(
        pl.kernel,
        out_type=jax.ShapeDtypeStruct((_NW * _LANES,), jnp.float32),
        mesh=mesh,
        compiler_params=pltpu.CompilerParams(needs_layout_passes=False),
        scratch_types=[
            pltpu.VMEM((L2,), jnp.int32),
            pltpu.VMEM((NF + _LANES,), jnp.float32),
            pltpu.VMEM((NF + _LANES,), jnp.float32),
            pltpu.VMEM((2 * NF,), jnp.float32),
            pltpu.VMEM((2 * NF,), jnp.float32),
            pltpu.VMEM((2 * L,), jnp.float32),
            pltpu.VMEM((2 * L,), jnp.float32),
            pltpu.VMEM((2 * CHUNK,), jnp.float32),
            pltpu.VMEM((2 * CHUNK,), jnp.float32),
            pltpu.VMEM((2 * CHUNK,), jnp.float32),
            pltpu.VMEM((2 * CHUNK,), jnp.float32),
            pltpu.VMEM((_LANES,), jnp.float32),
            pltpu.SemaphoreType.DMA,
            pltpu.SemaphoreType.DMA,
            pltpu.SemaphoreType.DMA,
        ],
    )
    def sc_call(cat, xi0, xi1, ks0, ks1, out,
                idxp, fpk, tpk, sta, stb, xis, kss,
                sci0, sci1, sct0, sct1, accv, sem_f, sem0, sem1):
        off_tf = 2 * NF
        off_is = 4 * NF
        off_ts = 4 * NF + 2 * L2
        cid = lax.axis_index("c")
        sid = lax.axis_index("s")
        wid = sid * _NUM_CORES + cid

        scis = (sci0, sci1)
        scts = (sct0, sct1)
        sems = (sem0, sem1)

        iota2 = lax.iota(jnp.int32, _LANES) * 2
        z16f = jnp.zeros((_LANES,), jnp.float32)

        fpk[pl.ds(NF, _LANES)] = z16f
        tpk[pl.ds(NF, _LANES)] = z16f

        for half, (xi, ks) in enumerate(((xi0, ks0), (xi1, ks1))):
            pltpu.sync_copy(xi, xis)
            pltpu.sync_copy(ks, kss)

            def pre_body(j, carry, half=half):
                for u in range(2):
                    lj = j * (2 * _LANES) + u * _LANES
                    le = 2 * lj + iota2
                    x0 = plsc.bitcast(plsc.load_gather(xis, [le]), jnp.int32)
                    x1 = plsc.bitcast(plsc.load_gather(xis, [le + 1]),
                                      jnp.int32)
                    k0 = plsc.load_gather(kss, [le])
                    k1 = plsc.load_gather(kss, [le + 1])
                    keep = jnp.logical_and(k0 <= 0.0, k1 <= 0.0)
                    i0m = jnp.where(keep, x0, zslot)
                    packed = jnp.bitwise_or(i0m, jnp.left_shift(x1, 16))
                    idxp[pl.ds(half * L + lj, _LANES)] = packed
                return carry

            lax.fori_loop(0, L // (2 * _LANES), pre_body, 0)

        def per_pair(j, acc):
            p = wid * bc_per_w + j
            b = p // C
            c = p % C

            d_f0 = pltpu.async_copy(
                cat.at[b, c, pl.ds(0, 2 * NF)], sta, sem_f)
            d_f1 = pltpu.async_copy(
                cat.at[b, c, pl.ds(off_tf, 2 * NF)], stb, sem_f)

            descs = {}

            def start_chunk(ch):
                par = ch % 2
                descs[ch] = (
                    pltpu.async_copy(
                        cat.at[b, c, pl.ds(off_is + 2 * ch * CHUNK,
                                           2 * CHUNK)],
                        scis[par], sems[par]),
                    pltpu.async_copy(
                        cat.at[b, c, pl.ds(off_ts + 2 * ch * CHUNK,
                                           2 * CHUNK)],
                        scts[par], sems[par]),
                )

            start_chunk(0)
            start_chunk(1)
            d_f0.wait()
            d_f1.wait()

            def tbl_body(n, carry):
                for u in range(2):
                    nj = n * (2 * _LANES) + u * _LANES
                    ne = 2 * nj + iota2
                    ife = plsc.load_gather(sta, [ne])
                    ifo = plsc.load_gather(sta, [ne + 1])
                    tfe = plsc.load_gather(stb, [ne])
                    tfo = plsc.load_gather(stb, [ne + 1])
                    sl = pl.ds(nj, _LANES)
                    fpk[sl] = plsc.bitcast(
                        plsc.pack(ife - tfe, ifo - tfo,
                                  format=plsc.PackFormat.INTERLEAVED),
                        jnp.float32)
                    tpk[sl] = plsc.bitcast(
                        plsc.pack(tfe, tfo,
                                  format=plsc.PackFormat.INTERLEAVED),
                        jnp.float32)
                return carry

            lax.fori_loop(0, NF // (2 * _LANES), tbl_body, 0)

            for ch in range(nch):
                par = ch % 2
                sci, sct = scis[par], scts[par]
                s_off = ch * CHUNK
                descs[ch][0].wait()
                descs[ch][1].wait()

                def inner(i, acc, sci=sci, sct=sct, s_off=s_off):
                    parts = []
                    for u in range(4):
                        li = i * (4 * _LANES) + u * _LANES
                        le = 2 * li + iota2
                        pk = idxp[pl.ds(s_off + li, _LANES)]
                        i0 = jnp.bitwise_and(pk, 0xFFFF)
                        i1 = lax.shift_right_logical(pk, 16)
                        fr0, fi0 = plsc.unpack(
                            plsc.bitcast(plsc.load_gather(fpk, [i0]),
                                         jnp.bfloat16),
                            format=plsc.PackFormat.INTERLEAVED)
                        tr0, ti0 = plsc.unpack(
                            plsc.bitcast(plsc.load_gather(tpk, [i0]),
                                         jnp.bfloat16),
                            format=plsc.PackFormat.INTERLEAVED)
                        fr1, fi1 = plsc.unpack(
                            plsc.bitcast(plsc.load_gather(fpk, [i1]),
                                         jnp.bfloat16),
                            format=plsc.PackFormat.INTERLEAVED)
                        tr1, ti1 = plsc.unpack(
                            plsc.bitcast(plsc.load_gather(tpk, [i1]),
                                         jnp.bfloat16),
                            format=plsc.PackFormat.INTERLEAVED)
                        sre = (plsc.load_gather(sci, [le])
                               - plsc.load_gather(sct, [le]))
                        sim = (plsc.load_gather(sci, [le + 1])
                               - plsc.load_gather(sct, [le + 1]))
                        ere = fr0 * tr1 + fi0 * ti1 + tr0 * fr1 + ti0 * fi1
                        eim = fi0 * tr1 - fr0 * ti1 + ti0 * fr1 - tr0 * fi1
                        gre = sre - ere
                        gim = sim - eim
                        parts.append(gre * gre + gim * gim)
                    return acc + ((parts[0] + parts[1])
                                  + (parts[2] + parts[3]))

                acc = lax.fori_loop(0, CHUNK // (4 * _LANES), inner, acc)
                if ch + 2 < nch:
                    start_chunk(ch + 2)

            return acc

        acc = lax.fori_loop(0, bc_per_w, per_pair, z16f)
        accv[...] = acc
        pltpu.sync_copy(accv, out.at[pl.ds(wid * _LANES, _LANES)])

    return sc_call


@jax.jit
def kernel(i_f, i_s, t_f, t_s, xi_idx0, xi_idx1, ks0, ks1):
    B, C, NF, _ = i_f.shape
    L = xi_idx0.shape[0]
    sc_call = _build_sc_call(B, C, NF, L)
    cat = jnp.concatenate(
        [i_f.reshape(B, C, 2 * NF), t_f.reshape(B, C, 2 * NF),
         i_s.reshape(B, C, 4 * L), t_s.reshape(B, C, 4 * L)], axis=-1)
    partials = sc_call(
        cat,
        lax.bitcast_convert_type(xi_idx0, jnp.float32).reshape(2 * L),
        lax.bitcast_convert_type(xi_idx1, jnp.float32).reshape(2 * L),
        ks0.reshape(2 * L),
        ks1.reshape(2 * L),
    )
    n = B * C * 2 * L
    return jnp.sum(partials) * jnp.float32(1.0 / n)
